# baseline (speedup 1.0000x reference)
import functools
import math

import jax
import jax.numpy as jnp
from jax import lax
from jax.experimental import pallas as pl
from jax.experimental.pallas import tpu as pltpu
from jax.experimental.pallas import tpu_sc as plsc

BT = 512
RB = 512
GCH = 64


def _gelu(v):
    return 0.5 * v * (1.0 + lax.erf(v * (1.0 / math.sqrt(2.0))))


def _router_body(x_ref, rw_ref, rb_ref, top_ref, gate_ref, rank_ref,
                 stats_ref, base_ref):
    i = pl.program_id(0)

    @pl.when(i == 0)
    def _init():
        base_ref[...] = jnp.zeros_like(base_ref)

    x = x_ref[...]
    logits = x @ rw_ref[...] + rb_ref[...]
    e = logits.shape[-1]
    m = jnp.max(logits, axis=1, keepdims=True)
    p = jnp.exp(logits - m)
    s = jnp.sum(p, axis=1, keepdims=True)
    gates = p / s

    lane = lax.broadcasted_iota(jnp.int32, logits.shape, 1)
    is_max = logits >= m
    top_i = jnp.min(jnp.where(is_max, lane, e), axis=1, keepdims=True)
    onehot = (lane == top_i).astype(jnp.float32)
    top_gate = jnp.sum(gates * onehot, axis=1, keepdims=True)

    n = x.shape[0]
    row = lax.broadcasted_iota(jnp.int32, (n, n), 0)
    col = lax.broadcasted_iota(jnp.int32, (n, n), 1)
    tri = (row > col).astype(jnp.float32)
    rank_mat = jax.lax.dot_general(tri, onehot, (((1,), (0,)), ((), ())),
                                   preferred_element_type=jnp.float32)
    rank_in = jnp.sum(rank_mat * onehot, axis=1, keepdims=True)
    base = base_ref[...]
    rank_g = rank_in + jnp.sum(onehot * base, axis=1, keepdims=True)

    counts_blk = jnp.sum(onehot, axis=0, keepdims=True)
    base_ref[...] = base + counts_blk

    top_ref[...] = top_i
    gate_ref[...] = top_gate
    rank_ref[...] = rank_g.astype(jnp.int32)
    stats_ref[0, 0:1, :] = jnp.sum(gates, axis=0, keepdims=True)
    stats_ref[0, 1:2, :] = counts_blk


def _run_router(x_flat, r_w, r_b):
    t, dim = x_flat.shape
    e = r_w.shape[-1]
    nb = t // RB
    grid = (nb,)
    out_shapes = (
        jax.ShapeDtypeStruct((t, 1), jnp.int32),
        jax.ShapeDtypeStruct((t, 1), jnp.float32),
        jax.ShapeDtypeStruct((t, 1), jnp.int32),
        jax.ShapeDtypeStruct((nb, 2, e), jnp.float32),
    )
    return pl.pallas_call(
        _router_body,
        grid=grid,
        in_specs=[
            pl.BlockSpec((RB, dim), lambda i: (i, 0)),
            pl.BlockSpec((dim, e), lambda i: (0, 0)),
            pl.BlockSpec((1, e), lambda i: (0, 0)),
        ],
        out_specs=(
            pl.BlockSpec((RB, 1), lambda i: (i, 0)),
            pl.BlockSpec((RB, 1), lambda i: (i, 0)),
            pl.BlockSpec((RB, 1), lambda i: (i, 0)),
            pl.BlockSpec((1, 2, e), lambda i: (i, 0, 0)),
        ),
        out_shape=out_shapes,
        scratch_shapes=[pltpu.VMEM((1, e), jnp.float32)],
    )(x_flat, r_w, r_b.reshape(1, e))


def _sc_gather_rows(idx, table, n_out):
    dim = table.shape[1]
    info = plsc.get_sparse_core_info()
    nw = info.num_cores * info.num_subcores
    bpw = n_out // nw
    nch = bpw // GCH
    mesh = plsc.VectorSubcoreMesh(core_axis_name="c", subcore_axis_name="s")

    @functools.partial(
        pl.kernel, mesh=mesh,
        out_type=jax.ShapeDtypeStruct((n_out, dim), jnp.float32),
        scratch_types=[
            pltpu.VMEM((GCH,), jnp.int32),
            pltpu.VMEM((GCH, dim), jnp.float32),
            pltpu.SemaphoreType.DMA,
        ],
    )
    def gather_k(idx_hbm, table_hbm, out_hbm, idx_v, rows_v, sem):
        wid = lax.axis_index("s") * info.num_cores + lax.axis_index("c")
        base = wid * bpw
        for c in range(nch):
            off = base + c * GCH
            pltpu.sync_copy(idx_hbm.at[pl.ds(off, GCH)], idx_v)
            pltpu.async_copy(table_hbm.at[idx_v], rows_v, sem).wait()
            pltpu.sync_copy(rows_v, out_hbm.at[pl.ds(off, GCH)])

    return gather_k(idx, table)


def _sc_scatter_rows(idx, rows, n_out):
    n_in, dim = rows.shape
    info = plsc.get_sparse_core_info()
    nw = info.num_cores * info.num_subcores
    bpw = n_in // nw
    nch = bpw // GCH
    mesh = plsc.VectorSubcoreMesh(core_axis_name="c", subcore_axis_name="s")

    @functools.partial(
        pl.kernel, mesh=mesh,
        out_type=jax.ShapeDtypeStruct((n_out, dim), jnp.float32),
        scratch_types=[
            pltpu.VMEM((GCH,), jnp.int32),
            pltpu.VMEM((GCH, dim), jnp.float32),
            pltpu.SemaphoreType.DMA,
        ],
    )
    def scatter_k(idx_hbm, rows_hbm, out_hbm, idx_v, rows_v, sem):
        wid = lax.axis_index("s") * info.num_cores + lax.axis_index("c")
        base = wid * bpw
        for c in range(nch):
            off = base + c * GCH
            pltpu.sync_copy(idx_hbm.at[pl.ds(off, GCH)], idx_v)
            pltpu.sync_copy(rows_hbm.at[pl.ds(off, GCH)], rows_v)
            pltpu.async_copy(rows_v, out_hbm.at[idx_v], sem).wait()

    return scatter_k(idx, rows)


def _experts_body(be_ref, xg_ref, w1_ref, b1_ref, w2_ref, b2_ref, out_ref):
    x = xg_ref[...]
    h = _gelu(jax.lax.dot_general(x, w1_ref[0], (((1,), (0,)), ((), ())),
                                  preferred_element_type=jnp.float32)
              + b1_ref[0])
    out_ref[...] = jax.lax.dot_general(h, w2_ref[0], (((1,), (0,)), ((), ())),
                                       preferred_element_type=jnp.float32) \
        + b2_ref[0]


def _run_experts(xg, block_expert, e_fc1_w, e_fc1_b, e_fc2_w, e_fc2_b):
    npad, dim = xg.shape
    e, _, hid = e_fc1_w.shape
    nblk = npad // BT
    grid_spec = pltpu.PrefetchScalarGridSpec(
        num_scalar_prefetch=1,
        grid=(nblk,),
        in_specs=[
            pl.BlockSpec((BT, dim), lambda i, be: (i, 0)),
            pl.BlockSpec((1, dim, hid), lambda i, be: (be[i], 0, 0)),
            pl.BlockSpec((1, 1, hid), lambda i, be: (be[i], 0, 0)),
            pl.BlockSpec((1, hid, dim), lambda i, be: (be[i], 0, 0)),
            pl.BlockSpec((1, 1, dim), lambda i, be: (be[i], 0, 0)),
        ],
        out_specs=pl.BlockSpec((BT, dim), lambda i, be: (i, 0)),
    )
    return pl.pallas_call(
        _experts_body,
        grid_spec=grid_spec,
        out_shape=jax.ShapeDtypeStruct((npad, dim), jnp.float32),
    )(block_expert, xg, e_fc1_w, e_fc1_b.reshape(e, 1, hid), e_fc2_w,
      e_fc2_b.reshape(e, 1, dim))


def _combine_body(sh_ref, yg_ref, g_ref, out_ref):
    out_ref[...] = sh_ref[...] + yg_ref[...] * g_ref[...]


def _run_combine(shared_y, yg, top_gate2):
    t, dim = shared_y.shape
    nb = t // RB
    return pl.pallas_call(
        _combine_body,
        grid=(nb,),
        in_specs=[
            pl.BlockSpec((RB, dim), lambda i: (i, 0)),
            pl.BlockSpec((RB, dim), lambda i: (i, 0)),
            pl.BlockSpec((RB, 1), lambda i: (i, 0)),
        ],
        out_specs=pl.BlockSpec((RB, dim), lambda i: (i, 0)),
        out_shape=jax.ShapeDtypeStruct((t, dim), jnp.float32),
    )(shared_y, yg, top_gate2)


def _shared_body(x_ref, w1_ref, b1_ref, w2_ref, b2_ref, out_ref):
    x = x_ref[...]
    acc = jnp.zeros_like(x)
    ns = w1_ref.shape[0]
    for i in range(ns):
        h = _gelu(jax.lax.dot_general(x, w1_ref[i], (((1,), (0,)), ((), ())),
                                      preferred_element_type=jnp.float32)
                  + b1_ref[i])
        acc = acc + jax.lax.dot_general(h, w2_ref[i], (((1,), (0,)), ((), ())),
                                        preferred_element_type=jnp.float32) \
            + b2_ref[i]
    out_ref[...] = acc


def _run_shared(x_flat, s_fc1_w, s_fc1_b, s_fc2_w, s_fc2_b):
    t, dim = x_flat.shape
    ns, _, hid = s_fc1_w.shape
    nb = t // RB
    return pl.pallas_call(
        _shared_body,
        grid=(nb,),
        in_specs=[
            pl.BlockSpec((RB, dim), lambda i: (i, 0)),
            pl.BlockSpec((ns, dim, hid), lambda i: (0, 0, 0)),
            pl.BlockSpec((ns, 1, hid), lambda i: (0, 0, 0)),
            pl.BlockSpec((ns, hid, dim), lambda i: (0, 0, 0)),
            pl.BlockSpec((ns, 1, dim), lambda i: (0, 0, 0)),
        ],
        out_specs=pl.BlockSpec((RB, dim), lambda i: (i, 0)),
        out_shape=jax.ShapeDtypeStruct((t, dim), jnp.float32),
    )(x_flat, s_fc1_w, s_fc1_b.reshape(ns, 1, hid), s_fc2_w,
      s_fc2_b.reshape(ns, 1, dim))


def kernel(x, s_fc1_w, s_fc1_b, s_fc2_w, s_fc2_b, e_fc1_w, e_fc1_b, e_fc2_w,
           e_fc2_b, r_w, r_b):
    b, hgt, wid_, c = x.shape
    t = b * hgt * wid_
    e = r_w.shape[-1]
    npad = t + e * BT
    x_flat = x.reshape(t, c)

    top_idx2, top_gate2, rank2, stats = _run_router(x_flat, r_w, r_b)
    top_idx = top_idx2[:, 0]
    top_gate = top_gate2[:, 0]
    rank = rank2[:, 0]

    imp_sum = stats[:, 0, :].sum(axis=0)
    counts = stats[:, 1, :].sum(axis=0)
    balance = jnp.sum((imp_sum / t) * (counts / t)) * e

    counts_i = counts.astype(jnp.int32)
    padded = ((counts_i + BT - 1) // BT) * BT
    ends = jnp.cumsum(padded)
    pad_off = ends - padded
    dest = pad_off[top_idx] + rank
    starts = jnp.arange(npad // BT, dtype=jnp.int32) * BT
    block_expert = jnp.minimum(
        jnp.searchsorted(ends, starts, side="right").astype(jnp.int32), e - 1)

    xg = _sc_scatter_rows(dest, x_flat, npad)
    shared_y = _run_shared(x_flat, s_fc1_w, s_fc1_b, s_fc2_w, s_fc2_b)
    y_pad = _run_experts(xg, block_expert, e_fc1_w, e_fc1_b, e_fc2_w, e_fc2_b)
    yg = _sc_gather_rows(dest, y_pad, t)

    out = _run_combine(shared_y, yg, top_gate2)
    return out.reshape(b, hgt, wid_, c), balance


# speedup vs baseline: 1.0190x; 1.0190x over previous
import functools
import math

import jax
import jax.numpy as jnp
from jax import lax
from jax.experimental import pallas as pl
from jax.experimental.pallas import tpu as pltpu
from jax.experimental.pallas import tpu_sc as plsc

BT = 512
RB = 512
GCH = 64


def _gelu(v):
    return 0.5 * v * (1.0 + lax.erf(v * (1.0 / math.sqrt(2.0))))


def _router_body(x_ref, rw_ref, rb_ref, top_ref, gate_ref, rank_ref,
                 stats_ref, base_ref):
    i = pl.program_id(0)

    @pl.when(i == 0)
    def _init():
        base_ref[...] = jnp.zeros_like(base_ref)

    x = x_ref[...]
    logits = x @ rw_ref[...] + rb_ref[...]
    e = logits.shape[-1]
    m = jnp.max(logits, axis=1, keepdims=True)
    p = jnp.exp(logits - m)
    s = jnp.sum(p, axis=1, keepdims=True)
    gates = p / s

    lane = lax.broadcasted_iota(jnp.int32, logits.shape, 1)
    is_max = logits >= m
    top_i = jnp.min(jnp.where(is_max, lane, e), axis=1, keepdims=True)
    onehot = (lane == top_i).astype(jnp.float32)
    top_gate = jnp.sum(gates * onehot, axis=1, keepdims=True)

    n = x.shape[0]
    row = lax.broadcasted_iota(jnp.int32, (n, n), 0)
    col = lax.broadcasted_iota(jnp.int32, (n, n), 1)
    tri = (row > col).astype(jnp.float32)
    rank_mat = jax.lax.dot_general(tri, onehot, (((1,), (0,)), ((), ())),
                                   preferred_element_type=jnp.float32)
    rank_in = jnp.sum(rank_mat * onehot, axis=1, keepdims=True)
    base = base_ref[...]
    rank_g = rank_in + jnp.sum(onehot * base, axis=1, keepdims=True)

    counts_blk = jnp.sum(onehot, axis=0, keepdims=True)
    base_ref[...] = base + counts_blk

    top_ref[...] = top_i
    gate_ref[...] = top_gate
    rank_ref[...] = rank_g.astype(jnp.int32)

    @pl.when(i == 0)
    def _zero_stats():
        stats_ref[...] = jnp.zeros_like(stats_ref)

    stats_ref[0:1, :] += jnp.sum(gates, axis=0, keepdims=True)
    stats_ref[1:2, :] += counts_blk


def _run_router(x_flat, r_w, r_b):
    t, dim = x_flat.shape
    e = r_w.shape[-1]
    nb = t // RB
    grid = (nb,)
    out_shapes = (
        jax.ShapeDtypeStruct((t, 1), jnp.int32),
        jax.ShapeDtypeStruct((t, 1), jnp.float32),
        jax.ShapeDtypeStruct((t, 1), jnp.int32),
        jax.ShapeDtypeStruct((2, e), jnp.float32),
    )
    return pl.pallas_call(
        _router_body,
        grid=grid,
        in_specs=[
            pl.BlockSpec((RB, dim), lambda i: (i, 0)),
            pl.BlockSpec((dim, e), lambda i: (0, 0)),
            pl.BlockSpec((1, e), lambda i: (0, 0)),
        ],
        out_specs=(
            pl.BlockSpec((RB, 1), lambda i: (i, 0)),
            pl.BlockSpec((RB, 1), lambda i: (i, 0)),
            pl.BlockSpec((RB, 1), lambda i: (i, 0)),
            pl.BlockSpec((2, e), lambda i: (0, 0)),
        ),
        out_shape=out_shapes,
        scratch_shapes=[pltpu.VMEM((1, e), jnp.float32)],
    )(x_flat, r_w, r_b.reshape(1, e))


def _sc_gather_rows(idx, table, n_out):
    dim = table.shape[1]
    info = plsc.get_sparse_core_info()
    nw = info.num_cores * info.num_subcores
    bpw = n_out // nw
    nch = bpw // GCH
    mesh = plsc.VectorSubcoreMesh(core_axis_name="c", subcore_axis_name="s")

    @functools.partial(
        pl.kernel, mesh=mesh,
        out_type=jax.ShapeDtypeStruct((n_out, dim), jnp.float32),
        scratch_types=[
            pltpu.VMEM((GCH,), jnp.int32),
            pltpu.VMEM((GCH, dim), jnp.float32),
            pltpu.SemaphoreType.DMA,
        ],
    )
    def gather_k(idx_hbm, table_hbm, out_hbm, idx_v, rows_v, sem):
        wid = lax.axis_index("s") * info.num_cores + lax.axis_index("c")
        base = wid * bpw
        for c in range(nch):
            off = base + c * GCH
            pltpu.sync_copy(idx_hbm.at[pl.ds(off, GCH)], idx_v)
            pltpu.async_copy(table_hbm.at[idx_v], rows_v, sem).wait()
            pltpu.sync_copy(rows_v, out_hbm.at[pl.ds(off, GCH)])

    return gather_k(idx, table)


def _sc_scatter_rows(idx, rows, n_out):
    n_in, dim = rows.shape
    info = plsc.get_sparse_core_info()
    nw = info.num_cores * info.num_subcores
    bpw = n_in // nw
    nch = bpw // GCH
    mesh = plsc.VectorSubcoreMesh(core_axis_name="c", subcore_axis_name="s")

    @functools.partial(
        pl.kernel, mesh=mesh,
        out_type=jax.ShapeDtypeStruct((n_out, dim), jnp.float32),
        scratch_types=[
            pltpu.VMEM((GCH,), jnp.int32),
            pltpu.VMEM((GCH, dim), jnp.float32),
            pltpu.SemaphoreType.DMA,
        ],
    )
    def scatter_k(idx_hbm, rows_hbm, out_hbm, idx_v, rows_v, sem):
        wid = lax.axis_index("s") * info.num_cores + lax.axis_index("c")
        base = wid * bpw
        for c in range(nch):
            off = base + c * GCH
            pltpu.sync_copy(idx_hbm.at[pl.ds(off, GCH)], idx_v)
            pltpu.sync_copy(rows_hbm.at[pl.ds(off, GCH)], rows_v)
            pltpu.async_copy(rows_v, out_hbm.at[idx_v], sem).wait()

    return scatter_k(idx, rows)


def _experts_body(be_ref, nu_ref, xg_ref, w1_ref, b1_ref, w2_ref, b2_ref,
                  out_ref):
    @pl.when(pl.program_id(0) * BT < nu_ref[0])
    def _work():
        x = xg_ref[...]
        h = _gelu(jax.lax.dot_general(x, w1_ref[0], (((1,), (0,)), ((), ())),
                                      preferred_element_type=jnp.float32)
                  + b1_ref[0])
        out_ref[...] = jax.lax.dot_general(h, w2_ref[0],
                                           (((1,), (0,)), ((), ())),
                                           preferred_element_type=jnp.float32)\
            + b2_ref[0]


def _run_experts(xg, block_expert, n_used, e_fc1_w, e_fc1_b, e_fc2_w,
                 e_fc2_b):
    npad, dim = xg.shape
    e, _, hid = e_fc1_w.shape
    nblk = npad // BT
    grid_spec = pltpu.PrefetchScalarGridSpec(
        num_scalar_prefetch=2,
        grid=(nblk,),
        in_specs=[
            pl.BlockSpec((BT, dim), lambda i, be, nu: (i, 0)),
            pl.BlockSpec((1, dim, hid), lambda i, be, nu: (be[i], 0, 0)),
            pl.BlockSpec((1, 1, hid), lambda i, be, nu: (be[i], 0, 0)),
            pl.BlockSpec((1, hid, dim), lambda i, be, nu: (be[i], 0, 0)),
            pl.BlockSpec((1, 1, dim), lambda i, be, nu: (be[i], 0, 0)),
        ],
        out_specs=pl.BlockSpec((BT, dim), lambda i, be, nu: (i, 0)),
    )
    return pl.pallas_call(
        _experts_body,
        grid_spec=grid_spec,
        out_shape=jax.ShapeDtypeStruct((npad, dim), jnp.float32),
    )(block_expert, n_used, xg, e_fc1_w, e_fc1_b.reshape(e, 1, hid), e_fc2_w,
      e_fc2_b.reshape(e, 1, dim))


def _combine_body(sha_ref, shb_ref, yg_ref, g_ref, out_ref):
    i = pl.program_id(0)
    nh = pl.num_programs(0) // 2
    sh = jnp.where(i < nh, sha_ref[...], shb_ref[...])
    out_ref[...] = sh + yg_ref[...] * g_ref[...]


def _run_combine(shared_a, shared_b, yg, top_gate2):
    t, dim = yg.shape
    nb = t // RB
    nh = nb // 2
    return pl.pallas_call(
        _combine_body,
        grid=(nb,),
        in_specs=[
            pl.BlockSpec((RB, dim), lambda i: (jnp.minimum(i, nh - 1), 0)),
            pl.BlockSpec((RB, dim),
                         lambda i: (jnp.clip(i - nh, 0, nh - 1), 0)),
            pl.BlockSpec((RB, dim), lambda i: (i, 0)),
            pl.BlockSpec((RB, 1), lambda i: (i, 0)),
        ],
        out_specs=pl.BlockSpec((RB, dim), lambda i: (i, 0)),
        out_shape=jax.ShapeDtypeStruct((t, dim), jnp.float32),
    )(shared_a, shared_b, yg, top_gate2)


def _shared_body(x_ref, w1_ref, b1_ref, w2_ref, b2_ref, out_ref):
    x = x_ref[...]
    acc = jnp.zeros_like(x)
    ns = w1_ref.shape[0]
    for i in range(ns):
        h = _gelu(jax.lax.dot_general(x, w1_ref[i], (((1,), (0,)), ((), ())),
                                      preferred_element_type=jnp.float32)
                  + b1_ref[i])
        acc = acc + jax.lax.dot_general(h, w2_ref[i], (((1,), (0,)), ((), ())),
                                        preferred_element_type=jnp.float32) \
            + b2_ref[i]
    out_ref[...] = acc


def _run_shared(x_flat, s_fc1_w, s_fc1_b, s_fc2_w, s_fc2_b, blk_off, nb):
    t, dim = x_flat.shape
    ns, _, hid = s_fc1_w.shape
    return pl.pallas_call(
        _shared_body,
        grid=(nb,),
        in_specs=[
            pl.BlockSpec((RB, dim), lambda i: (i + blk_off, 0)),
            pl.BlockSpec((ns, dim, hid), lambda i: (0, 0, 0)),
            pl.BlockSpec((ns, 1, hid), lambda i: (0, 0, 0)),
            pl.BlockSpec((ns, hid, dim), lambda i: (0, 0, 0)),
            pl.BlockSpec((ns, 1, dim), lambda i: (0, 0, 0)),
        ],
        out_specs=pl.BlockSpec((RB, dim), lambda i: (i, 0)),
        out_shape=jax.ShapeDtypeStruct((nb * RB, dim), jnp.float32),
    )(x_flat, s_fc1_w, s_fc1_b.reshape(ns, 1, hid), s_fc2_w,
      s_fc2_b.reshape(ns, 1, dim))


def kernel(x, s_fc1_w, s_fc1_b, s_fc2_w, s_fc2_b, e_fc1_w, e_fc1_b, e_fc2_w,
           e_fc2_b, r_w, r_b):
    b, hgt, wid_, c = x.shape
    t = b * hgt * wid_
    e = r_w.shape[-1]
    npad = t + e * BT
    x_flat = x.reshape(t, c)

    top_idx2, top_gate2, rank2, stats = _run_router(x_flat, r_w, r_b)
    top_idx = top_idx2[:, 0]
    top_gate = top_gate2[:, 0]
    rank = rank2[:, 0]

    imp_sum = stats[0]
    counts = stats[1]
    balance = jnp.sum((imp_sum / t) * (counts / t)) * e

    counts_i = counts.astype(jnp.int32)
    padded = ((counts_i + BT - 1) // BT) * BT
    ends = jnp.cumsum(padded)
    pad_off = ends - padded
    dest = pad_off[top_idx] + rank
    starts = jnp.arange(npad // BT, dtype=jnp.int32) * BT
    block_expert = jnp.minimum(
        jnp.searchsorted(ends, starts, side="right").astype(jnp.int32), e - 1)

    n_used = ends[e - 1:e]
    nb = t // RB
    xg = _sc_scatter_rows(dest, x_flat, npad)
    shared_a = _run_shared(x_flat, s_fc1_w, s_fc1_b, s_fc2_w, s_fc2_b,
                           0, nb // 2)
    y_pad = _run_experts(xg, block_expert, n_used, e_fc1_w, e_fc1_b, e_fc2_w,
                         e_fc2_b)
    shared_b = _run_shared(x_flat, s_fc1_w, s_fc1_b, s_fc2_w, s_fc2_b,
                           nb // 2, nb - nb // 2)
    yg = _sc_gather_rows(dest, y_pad, t)

    out = _run_combine(shared_a, shared_b, yg, top_gate2)
    return out.reshape(b, hgt, wid_, c), balance


# speedup vs baseline: 1.0824x; 1.0622x over previous
import functools
import math

import jax
import jax.numpy as jnp
from jax import lax
from jax.experimental import pallas as pl
from jax.experimental.pallas import tpu as pltpu
from jax.experimental.pallas import tpu_sc as plsc

BT = 512
RB = 512
GCH = 64


def _gelu(v):
    return 0.5 * v * (1.0 + lax.erf(v * (1.0 / math.sqrt(2.0))))


def _router_body(x_ref, rw_ref, rb_ref, top_ref, gate_ref, rank_ref,
                 stats_ref, base_ref):
    i = pl.program_id(0)

    @pl.when(i == 0)
    def _init():
        base_ref[...] = jnp.zeros_like(base_ref)

    x = x_ref[...]
    logits = x @ rw_ref[...] + rb_ref[...]
    e = logits.shape[-1]
    m = jnp.max(logits, axis=1, keepdims=True)
    p = jnp.exp(logits - m)
    s = jnp.sum(p, axis=1, keepdims=True)
    gates = p / s

    lane = lax.broadcasted_iota(jnp.int32, logits.shape, 1)
    is_max = logits >= m
    top_i = jnp.min(jnp.where(is_max, lane, e), axis=1, keepdims=True)
    onehot = (lane == top_i).astype(jnp.float32)
    top_gate = jnp.sum(gates * onehot, axis=1, keepdims=True)

    n = x.shape[0]
    row = lax.broadcasted_iota(jnp.int32, (n, n), 0)
    col = lax.broadcasted_iota(jnp.int32, (n, n), 1)
    tri = (row > col).astype(jnp.float32)
    rank_mat = jax.lax.dot_general(tri, onehot, (((1,), (0,)), ((), ())),
                                   preferred_element_type=jnp.float32)
    rank_in = jnp.sum(rank_mat * onehot, axis=1, keepdims=True)
    base = base_ref[...]
    rank_g = rank_in + jnp.sum(onehot * base, axis=1, keepdims=True)

    counts_blk = jnp.sum(onehot, axis=0, keepdims=True)
    base_ref[...] = base + counts_blk

    top_ref[...] = top_i
    gate_ref[...] = top_gate
    rank_ref[...] = rank_g.astype(jnp.int32)

    @pl.when(i == 0)
    def _zero_stats():
        stats_ref[...] = jnp.zeros_like(stats_ref)

    stats_ref[0:1, :] += jnp.sum(gates, axis=0, keepdims=True)
    stats_ref[1:2, :] += counts_blk


def _run_router(x_flat, r_w, r_b):
    t, dim = x_flat.shape
    e = r_w.shape[-1]
    nb = t // RB
    grid = (nb,)
    out_shapes = (
        jax.ShapeDtypeStruct((t, 1), jnp.int32),
        jax.ShapeDtypeStruct((t, 1), jnp.float32),
        jax.ShapeDtypeStruct((t, 1), jnp.int32),
        jax.ShapeDtypeStruct((2, e), jnp.float32),
    )
    return pl.pallas_call(
        _router_body,
        grid=grid,
        in_specs=[
            pl.BlockSpec((RB, dim), lambda i: (i, 0)),
            pl.BlockSpec((dim, e), lambda i: (0, 0)),
            pl.BlockSpec((1, e), lambda i: (0, 0)),
        ],
        out_specs=(
            pl.BlockSpec((RB, 1), lambda i: (i, 0)),
            pl.BlockSpec((RB, 1), lambda i: (i, 0)),
            pl.BlockSpec((RB, 1), lambda i: (i, 0)),
            pl.BlockSpec((2, e), lambda i: (0, 0)),
        ),
        out_shape=out_shapes,
        scratch_shapes=[pltpu.VMEM((1, e), jnp.float32)],
    )(x_flat, r_w, r_b.reshape(1, e))


def _sc_gather_rows(idx, table, n_out):
    dim = table.shape[1]
    info = plsc.get_sparse_core_info()
    nw = info.num_cores * info.num_subcores
    bpw = n_out // nw
    nch = bpw // GCH
    mesh = plsc.VectorSubcoreMesh(core_axis_name="c", subcore_axis_name="s")

    @functools.partial(
        pl.kernel, mesh=mesh,
        out_type=jax.ShapeDtypeStruct((n_out, dim), jnp.float32),
        scratch_types=[
            pltpu.VMEM((GCH,), jnp.int32),
            pltpu.VMEM((GCH, dim), jnp.float32),
            pltpu.SemaphoreType.DMA,
        ],
    )
    def gather_k(idx_hbm, table_hbm, out_hbm, idx_v, rows_v, sem):
        wid = lax.axis_index("s") * info.num_cores + lax.axis_index("c")
        base = wid * bpw
        for c in range(nch):
            off = base + c * GCH
            pltpu.sync_copy(idx_hbm.at[pl.ds(off, GCH)], idx_v)
            pltpu.async_copy(table_hbm.at[idx_v], rows_v, sem).wait()
            pltpu.sync_copy(rows_v, out_hbm.at[pl.ds(off, GCH)])

    return gather_k(idx, table)


def _sc_scatter_rows(idx, rows, n_out):
    n_in, dim = rows.shape
    info = plsc.get_sparse_core_info()
    nw = info.num_cores * info.num_subcores
    bpw = n_in // nw
    nch = bpw // GCH
    mesh = plsc.VectorSubcoreMesh(core_axis_name="c", subcore_axis_name="s")

    @functools.partial(
        pl.kernel, mesh=mesh,
        out_type=jax.ShapeDtypeStruct((n_out, dim), jnp.float32),
        scratch_types=[
            pltpu.VMEM((GCH,), jnp.int32),
            pltpu.VMEM((GCH, dim), jnp.float32),
            pltpu.SemaphoreType.DMA,
        ],
    )
    def scatter_k(idx_hbm, rows_hbm, out_hbm, idx_v, rows_v, sem):
        wid = lax.axis_index("s") * info.num_cores + lax.axis_index("c")
        base = wid * bpw
        for c in range(nch):
            off = base + c * GCH
            pltpu.sync_copy(idx_hbm.at[pl.ds(off, GCH)], idx_v)
            pltpu.sync_copy(rows_hbm.at[pl.ds(off, GCH)], rows_v)
            pltpu.async_copy(rows_v, out_hbm.at[idx_v], sem).wait()

    return scatter_k(idx, rows)


def _experts_body(be_ref, nu_ref, xg_ref, w1_ref, b1_ref, w2_ref, b2_ref,
                  out_ref):
    @pl.when(pl.program_id(0) * BT < nu_ref[0])
    def _work():
        x = xg_ref[...]
        h = _gelu(jax.lax.dot_general(x, w1_ref[0], (((1,), (0,)), ((), ())),
                                      preferred_element_type=jnp.float32)
                  + b1_ref[0])
        out_ref[...] = jax.lax.dot_general(h, w2_ref[0],
                                           (((1,), (0,)), ((), ())),
                                           preferred_element_type=jnp.float32)\
            + b2_ref[0]


def _run_experts(xg, block_expert, n_used, e_fc1_w, e_fc1_b, e_fc2_w,
                 e_fc2_b):
    npad, dim = xg.shape
    e, _, hid = e_fc1_w.shape
    nblk = npad // BT
    grid_spec = pltpu.PrefetchScalarGridSpec(
        num_scalar_prefetch=2,
        grid=(nblk,),
        in_specs=[
            pl.BlockSpec((BT, dim), lambda i, be, nu: (i, 0)),
            pl.BlockSpec((1, dim, hid), lambda i, be, nu: (be[i], 0, 0)),
            pl.BlockSpec((1, 1, hid), lambda i, be, nu: (be[i], 0, 0)),
            pl.BlockSpec((1, hid, dim), lambda i, be, nu: (be[i], 0, 0)),
            pl.BlockSpec((1, 1, dim), lambda i, be, nu: (be[i], 0, 0)),
        ],
        out_specs=pl.BlockSpec((BT, dim), lambda i, be, nu: (i, 0)),
    )
    return pl.pallas_call(
        _experts_body,
        grid_spec=grid_spec,
        out_shape=jax.ShapeDtypeStruct((npad, dim), jnp.float32),
    )(block_expert, n_used, xg, e_fc1_w, e_fc1_b.reshape(e, 1, hid), e_fc2_w,
      e_fc2_b.reshape(e, 1, dim))


def _combine_body(sha_ref, shb_ref, yg_ref, g_ref, out_ref):
    i = pl.program_id(0)
    nh = pl.num_programs(0) // 2
    sh = jnp.where(i < nh, sha_ref[...], shb_ref[...])
    out_ref[...] = sh + yg_ref[...] * g_ref[...]


def _run_combine(shared_a, shared_b, yg, top_gate2):
    t, dim = yg.shape
    nb = t // RB
    nh = nb // 2
    return pl.pallas_call(
        _combine_body,
        grid=(nb,),
        in_specs=[
            pl.BlockSpec((RB, dim), lambda i: (jnp.minimum(i, nh - 1), 0)),
            pl.BlockSpec((RB, dim),
                         lambda i: (jnp.clip(i - nh, 0, nh - 1), 0)),
            pl.BlockSpec((RB, dim), lambda i: (i, 0)),
            pl.BlockSpec((RB, 1), lambda i: (i, 0)),
        ],
        out_specs=pl.BlockSpec((RB, dim), lambda i: (i, 0)),
        out_shape=jax.ShapeDtypeStruct((t, dim), jnp.float32),
    )(shared_a, shared_b, yg, top_gate2)


def _shared_body(x_ref, w1_ref, b1_ref, w2_ref, b2_ref, out_ref):
    x = x_ref[...]
    acc = jnp.zeros_like(x)
    ns = w1_ref.shape[0]
    for i in range(ns):
        h = _gelu(jax.lax.dot_general(x, w1_ref[i], (((1,), (0,)), ((), ())),
                                      preferred_element_type=jnp.float32)
                  + b1_ref[i])
        acc = acc + jax.lax.dot_general(h, w2_ref[i], (((1,), (0,)), ((), ())),
                                        preferred_element_type=jnp.float32) \
            + b2_ref[i]
    out_ref[...] = acc


def _run_shared(x_flat, s_fc1_w, s_fc1_b, s_fc2_w, s_fc2_b, blk_off, nb):
    t, dim = x_flat.shape
    ns, _, hid = s_fc1_w.shape
    return pl.pallas_call(
        _shared_body,
        grid=(nb,),
        in_specs=[
            pl.BlockSpec((RB, dim), lambda i: (i + blk_off, 0)),
            pl.BlockSpec((ns, dim, hid), lambda i: (0, 0, 0)),
            pl.BlockSpec((ns, 1, hid), lambda i: (0, 0, 0)),
            pl.BlockSpec((ns, hid, dim), lambda i: (0, 0, 0)),
            pl.BlockSpec((ns, 1, dim), lambda i: (0, 0, 0)),
        ],
        out_specs=pl.BlockSpec((RB, dim), lambda i: (i, 0)),
        out_shape=jax.ShapeDtypeStruct((nb * RB, dim), jnp.float32),
    )(x_flat, s_fc1_w, s_fc1_b.reshape(ns, 1, hid), s_fc2_w,
      s_fc2_b.reshape(ns, 1, dim))


def kernel(x, s_fc1_w, s_fc1_b, s_fc2_w, s_fc2_b, e_fc1_w, e_fc1_b, e_fc2_w,
           e_fc2_b, r_w, r_b):
    b, hgt, wid_, c = x.shape
    t = b * hgt * wid_
    e = r_w.shape[-1]
    npad = t + e * BT
    x_flat = x.reshape(t, c)

    top_idx2, top_gate2, rank2, stats = _run_router(x_flat, r_w, r_b)
    top_idx = top_idx2[:, 0]
    top_gate = top_gate2[:, 0]
    rank = rank2[:, 0]

    imp_sum = stats[0]
    counts = stats[1]
    balance = jnp.sum((imp_sum / t) * (counts / t)) * e

    counts_i = counts.astype(jnp.int32)
    padded = ((counts_i + BT - 1) // BT) * BT
    ends = jnp.cumsum(padded)
    pad_off = ends - padded
    dest = pad_off[top_idx] + rank
    starts = jnp.arange(npad // BT, dtype=jnp.int32) * BT
    block_expert = jnp.minimum(
        jnp.searchsorted(ends, starts, side="right").astype(jnp.int32), e - 1)

    n_used = ends[e - 1:e]
    nb = t // RB
    xg = _sc_scatter_rows(dest, x_flat, npad)
    shared_a = _run_shared(x_flat, s_fc1_w, s_fc1_b, s_fc2_w, s_fc2_b,
                           0, nb // 2)
    n_used, shared_a = jax.lax.optimization_barrier((n_used, shared_a))
    y_pad = _run_experts(xg, block_expert, n_used, e_fc1_w, e_fc1_b, e_fc2_w,
                         e_fc2_b)
    shared_b = _run_shared(x_flat, s_fc1_w, s_fc1_b, s_fc2_w, s_fc2_b,
                           nb // 2, nb - nb // 2)
    yg = _sc_gather_rows(dest, y_pad, t)

    out = _run_combine(shared_a, shared_b, yg, top_gate2)
    return out.reshape(b, hgt, wid_, c), balance


# speedup vs baseline: 1.1480x; 1.0606x over previous
import functools
import math

import jax
import jax.numpy as jnp
from jax import lax
from jax.experimental import pallas as pl
from jax.experimental.pallas import tpu as pltpu
from jax.experimental.pallas import tpu_sc as plsc

BT = 512
RB = 512
GCH = 64


def _gelu(v):
    return 0.5 * v * (1.0 + lax.erf(v * (1.0 / math.sqrt(2.0))))


def _router_body(x_ref, rw_ref, rb_ref, top_ref, gate_ref, rank_ref,
                 stats_ref, base_ref):
    i = pl.program_id(0)

    @pl.when(i == 0)
    def _init():
        base_ref[...] = jnp.zeros_like(base_ref)

    x = x_ref[...]
    logits = x @ rw_ref[...] + rb_ref[...]
    e = logits.shape[-1]
    m = jnp.max(logits, axis=1, keepdims=True)
    p = jnp.exp(logits - m)
    s = jnp.sum(p, axis=1, keepdims=True)
    gates = p / s

    lane = lax.broadcasted_iota(jnp.int32, logits.shape, 1)
    is_max = logits >= m
    top_i = jnp.min(jnp.where(is_max, lane, e), axis=1, keepdims=True)
    onehot = (lane == top_i).astype(jnp.float32)
    top_gate = jnp.sum(gates * onehot, axis=1, keepdims=True)

    n = x.shape[0]
    row = lax.broadcasted_iota(jnp.int32, (n, n), 0)
    col = lax.broadcasted_iota(jnp.int32, (n, n), 1)
    tri = (row > col).astype(jnp.float32)
    rank_mat = jax.lax.dot_general(tri, onehot, (((1,), (0,)), ((), ())),
                                   preferred_element_type=jnp.float32)
    rank_in = jnp.sum(rank_mat * onehot, axis=1, keepdims=True)
    base = base_ref[...]
    rank_g = rank_in + jnp.sum(onehot * base, axis=1, keepdims=True)

    counts_blk = jnp.sum(onehot, axis=0, keepdims=True)
    base_ref[...] = base + counts_blk

    top_ref[...] = top_i
    gate_ref[...] = top_gate
    rank_ref[...] = rank_g.astype(jnp.int32)

    @pl.when(i == 0)
    def _zero_stats():
        stats_ref[...] = jnp.zeros_like(stats_ref)

    stats_ref[0:1, :] += jnp.sum(gates, axis=0, keepdims=True)
    stats_ref[1:2, :] += counts_blk


def _run_router(x_flat, r_w, r_b):
    t, dim = x_flat.shape
    e = r_w.shape[-1]
    nb = t // RB
    grid = (nb,)
    out_shapes = (
        jax.ShapeDtypeStruct((t, 1), jnp.int32),
        jax.ShapeDtypeStruct((t, 1), jnp.float32),
        jax.ShapeDtypeStruct((t, 1), jnp.int32),
        jax.ShapeDtypeStruct((2, e), jnp.float32),
    )
    return pl.pallas_call(
        _router_body,
        grid=grid,
        in_specs=[
            pl.BlockSpec((RB, dim), lambda i: (i, 0)),
            pl.BlockSpec((dim, e), lambda i: (0, 0)),
            pl.BlockSpec((1, e), lambda i: (0, 0)),
        ],
        out_specs=(
            pl.BlockSpec((RB, 1), lambda i: (i, 0)),
            pl.BlockSpec((RB, 1), lambda i: (i, 0)),
            pl.BlockSpec((RB, 1), lambda i: (i, 0)),
            pl.BlockSpec((2, e), lambda i: (0, 0)),
        ),
        out_shape=out_shapes,
        scratch_shapes=[pltpu.VMEM((1, e), jnp.float32)],
    )(x_flat, r_w, r_b.reshape(1, e))


def _prep_body(top_ref, rank_ref, stats_ref, dest_ref, be_ref, nu_ref,
               bal_ref):
    e = stats_ref.shape[1]
    t = top_ref.shape[0] * top_ref.shape[1]
    counts = stats_ref[1:2, :]
    padded = jnp.floor((counts + (BT - 1)) * (1.0 / BT)) * BT
    lo = lax.broadcasted_iota(jnp.int32, (e, e), 0)
    hi = lax.broadcasted_iota(jnp.int32, (e, e), 1)
    incl = (lo <= hi).astype(jnp.float32)
    ends = jax.lax.dot_general(padded, incl, (((1,), (0,)), ((), ())),
                               preferred_element_type=jnp.float32)
    pad_off = ends - padded

    top = top_ref[...]
    off = jnp.zeros(top.shape, jnp.float32)
    for k in range(e):
        off = off + jnp.where(top == k, pad_off[0:1, k:k + 1], 0.0)
    dest_ref[...] = rank_ref[...] + off.astype(jnp.int32)

    nblk = be_ref.shape[1]
    starts = lax.broadcasted_iota(jnp.int32, (1, nblk), 1) * BT
    acc = jnp.zeros((1, nblk), jnp.int32)
    for k in range(e - 1):
        acc = acc + (starts.astype(jnp.float32) >= ends[0:1, k:k + 1]
                     ).astype(jnp.int32)
    be_ref[...] = acc
    nu_ref[...] = ends[0:1, e - 1:e].astype(jnp.int32)
    bal_ref[...] = jnp.sum(stats_ref[0:1, :] * counts, keepdims=True) \
        * (float(e) / (float(t) * float(t)))


def _run_prep(top2, rank2, stats, nblk):
    t = top2.shape[0]
    e = stats.shape[1]
    top_r = top2.reshape(t // 128, 128)
    rank_r = rank2.reshape(t // 128, 128)
    out = pl.pallas_call(
        _prep_body,
        out_shape=(
            jax.ShapeDtypeStruct((t // 128, 128), jnp.int32),
            jax.ShapeDtypeStruct((1, nblk), jnp.int32),
            jax.ShapeDtypeStruct((1, 1), jnp.int32),
            jax.ShapeDtypeStruct((1, 1), jnp.float32),
        ),
    )(top_r, rank_r, stats)
    dest, be, nu, bal = out
    return (dest.reshape(t), be.reshape(nblk), nu.reshape(1),
            bal.reshape(()))


def _sc_gather_rows(idx, table, n_out):
    dim = table.shape[1]
    info = plsc.get_sparse_core_info()
    nw = info.num_cores * info.num_subcores
    bpw = n_out // nw
    nch = bpw // GCH
    mesh = plsc.VectorSubcoreMesh(core_axis_name="c", subcore_axis_name="s")

    @functools.partial(
        pl.kernel, mesh=mesh,
        out_type=jax.ShapeDtypeStruct((n_out, dim), jnp.float32),
        scratch_types=[
            pltpu.VMEM((GCH,), jnp.int32),
            pltpu.VMEM((GCH, dim), jnp.float32),
            pltpu.SemaphoreType.DMA,
        ],
    )
    def gather_k(idx_hbm, table_hbm, out_hbm, idx_v, rows_v, sem):
        wid = lax.axis_index("s") * info.num_cores + lax.axis_index("c")
        base = wid * bpw
        for c in range(nch):
            off = base + c * GCH
            pltpu.sync_copy(idx_hbm.at[pl.ds(off, GCH)], idx_v)
            pltpu.async_copy(table_hbm.at[idx_v], rows_v, sem).wait()
            pltpu.sync_copy(rows_v, out_hbm.at[pl.ds(off, GCH)])

    return gather_k(idx, table)


def _sc_scatter_rows(idx, rows, n_out):
    n_in, dim = rows.shape
    info = plsc.get_sparse_core_info()
    nw = info.num_cores * info.num_subcores
    bpw = n_in // nw
    nch = bpw // GCH
    mesh = plsc.VectorSubcoreMesh(core_axis_name="c", subcore_axis_name="s")

    @functools.partial(
        pl.kernel, mesh=mesh,
        out_type=jax.ShapeDtypeStruct((n_out, dim), jnp.float32),
        scratch_types=[
            pltpu.VMEM((GCH,), jnp.int32),
            pltpu.VMEM((GCH, dim), jnp.float32),
            pltpu.SemaphoreType.DMA,
        ],
    )
    def scatter_k(idx_hbm, rows_hbm, out_hbm, idx_v, rows_v, sem):
        wid = lax.axis_index("s") * info.num_cores + lax.axis_index("c")
        base = wid * bpw
        for c in range(nch):
            off = base + c * GCH
            pltpu.sync_copy(idx_hbm.at[pl.ds(off, GCH)], idx_v)
            pltpu.sync_copy(rows_hbm.at[pl.ds(off, GCH)], rows_v)
            pltpu.async_copy(rows_v, out_hbm.at[idx_v], sem).wait()

    return scatter_k(idx, rows)


def _experts_body(be_ref, nu_ref, xg_ref, w1_ref, b1_ref, w2_ref, b2_ref,
                  out_ref):
    @pl.when(pl.program_id(0) * BT < nu_ref[0])
    def _work():
        x = xg_ref[...]
        h = _gelu(jax.lax.dot_general(x, w1_ref[0], (((1,), (0,)), ((), ())),
                                      preferred_element_type=jnp.float32)
                  + b1_ref[0])
        out_ref[...] = jax.lax.dot_general(h, w2_ref[0],
                                           (((1,), (0,)), ((), ())),
                                           preferred_element_type=jnp.float32)\
            + b2_ref[0]


def _run_experts(xg, block_expert, n_used, e_fc1_w, e_fc1_b, e_fc2_w,
                 e_fc2_b):
    npad, dim = xg.shape
    e, _, hid = e_fc1_w.shape
    nblk = npad // BT
    grid_spec = pltpu.PrefetchScalarGridSpec(
        num_scalar_prefetch=2,
        grid=(nblk,),
        in_specs=[
            pl.BlockSpec((BT, dim), lambda i, be, nu: (i, 0)),
            pl.BlockSpec((1, dim, hid), lambda i, be, nu: (be[i], 0, 0)),
            pl.BlockSpec((1, 1, hid), lambda i, be, nu: (be[i], 0, 0)),
            pl.BlockSpec((1, hid, dim), lambda i, be, nu: (be[i], 0, 0)),
            pl.BlockSpec((1, 1, dim), lambda i, be, nu: (be[i], 0, 0)),
        ],
        out_specs=pl.BlockSpec((BT, dim), lambda i, be, nu: (i, 0)),
    )
    return pl.pallas_call(
        _experts_body,
        grid_spec=grid_spec,
        out_shape=jax.ShapeDtypeStruct((npad, dim), jnp.float32),
    )(block_expert, n_used, xg, e_fc1_w, e_fc1_b.reshape(e, 1, hid), e_fc2_w,
      e_fc2_b.reshape(e, 1, dim))


def _combine_body(sha_ref, shb_ref, yg_ref, g_ref, out_ref):
    i = pl.program_id(0)
    nh = pl.num_programs(0) // 2
    sh = jnp.where(i < nh, sha_ref[...], shb_ref[...])
    out_ref[...] = sh + yg_ref[...] * g_ref[...]


def _run_combine(shared_a, shared_b, yg, top_gate2):
    t, dim = yg.shape
    nb = t // RB
    nh = nb // 2
    return pl.pallas_call(
        _combine_body,
        grid=(nb,),
        in_specs=[
            pl.BlockSpec((RB, dim), lambda i: (jnp.minimum(i, nh - 1), 0)),
            pl.BlockSpec((RB, dim),
                         lambda i: (jnp.clip(i - nh, 0, nh - 1), 0)),
            pl.BlockSpec((RB, dim), lambda i: (i, 0)),
            pl.BlockSpec((RB, 1), lambda i: (i, 0)),
        ],
        out_specs=pl.BlockSpec((RB, dim), lambda i: (i, 0)),
        out_shape=jax.ShapeDtypeStruct((t, dim), jnp.float32),
    )(shared_a, shared_b, yg, top_gate2)


def _shared_body(x_ref, w1_ref, b1_ref, w2_ref, b2_ref, out_ref):
    x = x_ref[...]
    acc = jnp.zeros_like(x)
    ns = w1_ref.shape[0]
    for i in range(ns):
        h = _gelu(jax.lax.dot_general(x, w1_ref[i], (((1,), (0,)), ((), ())),
                                      preferred_element_type=jnp.float32)
                  + b1_ref[i])
        acc = acc + jax.lax.dot_general(h, w2_ref[i], (((1,), (0,)), ((), ())),
                                        preferred_element_type=jnp.float32) \
            + b2_ref[i]
    out_ref[...] = acc


def _run_shared(x_flat, s_fc1_w, s_fc1_b, s_fc2_w, s_fc2_b, blk_off, nb):
    t, dim = x_flat.shape
    ns, _, hid = s_fc1_w.shape
    return pl.pallas_call(
        _shared_body,
        grid=(nb,),
        in_specs=[
            pl.BlockSpec((RB, dim), lambda i: (i + blk_off, 0)),
            pl.BlockSpec((ns, dim, hid), lambda i: (0, 0, 0)),
            pl.BlockSpec((ns, 1, hid), lambda i: (0, 0, 0)),
            pl.BlockSpec((ns, hid, dim), lambda i: (0, 0, 0)),
            pl.BlockSpec((ns, 1, dim), lambda i: (0, 0, 0)),
        ],
        out_specs=pl.BlockSpec((RB, dim), lambda i: (i, 0)),
        out_shape=jax.ShapeDtypeStruct((nb * RB, dim), jnp.float32),
    )(x_flat, s_fc1_w, s_fc1_b.reshape(ns, 1, hid), s_fc2_w,
      s_fc2_b.reshape(ns, 1, dim))


def kernel(x, s_fc1_w, s_fc1_b, s_fc2_w, s_fc2_b, e_fc1_w, e_fc1_b, e_fc2_w,
           e_fc2_b, r_w, r_b):
    b, hgt, wid_, c = x.shape
    t = b * hgt * wid_
    e = r_w.shape[-1]
    npad = t + e * BT
    x_flat = x.reshape(t, c)

    top_idx2, top_gate2, rank2, stats = _run_router(x_flat, r_w, r_b)

    dest, block_expert, n_used, balance = _run_prep(top_idx2, rank2, stats,
                                                    npad // BT)

    nb = t // RB
    xg = _sc_scatter_rows(dest, x_flat, npad)
    shared_a = _run_shared(x_flat, s_fc1_w, s_fc1_b, s_fc2_w, s_fc2_b,
                           0, nb // 2)
    n_used, shared_a = jax.lax.optimization_barrier((n_used, shared_a))
    y_pad = _run_experts(xg, block_expert, n_used, e_fc1_w, e_fc1_b, e_fc2_w,
                         e_fc2_b)
    shared_b = _run_shared(x_flat, s_fc1_w, s_fc1_b, s_fc2_w, s_fc2_b,
                           nb // 2, nb - nb // 2)
    yg = _sc_gather_rows(dest, y_pad, t)

    out = _run_combine(shared_a, shared_b, yg, top_gate2)
    return out.reshape(b, hgt, wid_, c), balance


# speedup vs baseline: 1.1784x; 1.0265x over previous
import functools
import math

import jax
import jax.numpy as jnp
from jax import lax
from jax.experimental import pallas as pl
from jax.experimental.pallas import tpu as pltpu
from jax.experimental.pallas import tpu_sc as plsc

BT = 512
RB = 1024
GCH = 64


def _gelu(v):
    return 0.5 * v * (1.0 + lax.erf(v * (1.0 / math.sqrt(2.0))))


def _router_body(x_ref, rw_ref, rb_ref, top_ref, gate_ref, rank_ref,
                 stats_ref, base_ref):
    i = pl.program_id(0)

    @pl.when(i == 0)
    def _init():
        base_ref[...] = jnp.zeros_like(base_ref)

    x = x_ref[...]
    logits = x @ rw_ref[...] + rb_ref[...]
    e = logits.shape[-1]
    m = jnp.max(logits, axis=1, keepdims=True)
    p = jnp.exp(logits - m)
    s = jnp.sum(p, axis=1, keepdims=True)
    gates = p / s

    lane = lax.broadcasted_iota(jnp.int32, logits.shape, 1)
    is_max = logits >= m
    top_i = jnp.min(jnp.where(is_max, lane, e), axis=1, keepdims=True)
    onehot = (lane == top_i).astype(jnp.float32)
    top_gate = jnp.sum(gates * onehot, axis=1, keepdims=True)

    n = x.shape[0]
    row = lax.broadcasted_iota(jnp.int32, (n, n), 0)
    col = lax.broadcasted_iota(jnp.int32, (n, n), 1)
    tri = (row > col).astype(jnp.float32)
    rank_mat = jax.lax.dot_general(tri, onehot, (((1,), (0,)), ((), ())),
                                   preferred_element_type=jnp.float32)
    rank_in = jnp.sum(rank_mat * onehot, axis=1, keepdims=True)
    base = base_ref[...]
    rank_g = rank_in + jnp.sum(onehot * base, axis=1, keepdims=True)

    counts_blk = jnp.sum(onehot, axis=0, keepdims=True)
    base_ref[...] = base + counts_blk

    top_ref[...] = top_i
    gate_ref[...] = top_gate
    rank_ref[...] = rank_g.astype(jnp.int32)

    @pl.when(i == 0)
    def _zero_stats():
        stats_ref[...] = jnp.zeros_like(stats_ref)

    stats_ref[0:1, :] += jnp.sum(gates, axis=0, keepdims=True)
    stats_ref[1:2, :] += counts_blk


def _run_router(x_flat, r_w, r_b):
    t, dim = x_flat.shape
    e = r_w.shape[-1]
    nb = t // RB
    grid = (nb,)
    out_shapes = (
        jax.ShapeDtypeStruct((t, 1), jnp.int32),
        jax.ShapeDtypeStruct((t, 1), jnp.float32),
        jax.ShapeDtypeStruct((t, 1), jnp.int32),
        jax.ShapeDtypeStruct((2, e), jnp.float32),
    )
    return pl.pallas_call(
        _router_body,
        grid=grid,
        in_specs=[
            pl.BlockSpec((RB, dim), lambda i: (i, 0)),
            pl.BlockSpec((dim, e), lambda i: (0, 0)),
            pl.BlockSpec((1, e), lambda i: (0, 0)),
        ],
        out_specs=(
            pl.BlockSpec((RB, 1), lambda i: (i, 0)),
            pl.BlockSpec((RB, 1), lambda i: (i, 0)),
            pl.BlockSpec((RB, 1), lambda i: (i, 0)),
            pl.BlockSpec((2, e), lambda i: (0, 0)),
        ),
        out_shape=out_shapes,
        scratch_shapes=[pltpu.VMEM((1, e), jnp.float32)],
    )(x_flat, r_w, r_b.reshape(1, e))


def _prep_body(top_ref, rank_ref, stats_ref, dest_ref, be_ref, nu_ref,
               bal_ref):
    e = stats_ref.shape[1]
    t = top_ref.shape[0] * top_ref.shape[1]
    counts = stats_ref[1:2, :]
    padded = jnp.floor((counts + (BT - 1)) * (1.0 / BT)) * BT
    lo = lax.broadcasted_iota(jnp.int32, (e, e), 0)
    hi = lax.broadcasted_iota(jnp.int32, (e, e), 1)
    incl = (lo <= hi).astype(jnp.float32)
    ends = jax.lax.dot_general(padded, incl, (((1,), (0,)), ((), ())),
                               preferred_element_type=jnp.float32)
    pad_off = ends - padded

    top = top_ref[...]
    off = jnp.zeros(top.shape, jnp.float32)
    for k in range(e):
        off = off + jnp.where(top == k, pad_off[0:1, k:k + 1], 0.0)
    dest_ref[...] = rank_ref[...] + off.astype(jnp.int32)

    nblk = be_ref.shape[1]
    starts = lax.broadcasted_iota(jnp.int32, (1, nblk), 1) * BT
    acc = jnp.zeros((1, nblk), jnp.int32)
    for k in range(e - 1):
        acc = acc + (starts.astype(jnp.float32) >= ends[0:1, k:k + 1]
                     ).astype(jnp.int32)
    be_ref[...] = acc
    nu_ref[...] = ends[0:1, e - 1:e].astype(jnp.int32)
    bal_ref[...] = jnp.sum(stats_ref[0:1, :] * counts, keepdims=True) \
        * (float(e) / (float(t) * float(t)))


def _run_prep(top2, rank2, stats, nblk):
    t = top2.shape[0]
    e = stats.shape[1]
    top_r = top2.reshape(t // 128, 128)
    rank_r = rank2.reshape(t // 128, 128)
    out = pl.pallas_call(
        _prep_body,
        out_shape=(
            jax.ShapeDtypeStruct((t // 128, 128), jnp.int32),
            jax.ShapeDtypeStruct((1, nblk), jnp.int32),
            jax.ShapeDtypeStruct((1, 1), jnp.int32),
            jax.ShapeDtypeStruct((1, 1), jnp.float32),
        ),
    )(top_r, rank_r, stats)
    dest, be, nu, bal = out
    return (dest.reshape(t), be.reshape(nblk), nu.reshape(1),
            bal.reshape(()))


def _sc_gather_rows(idx, table, n_out):
    dim = table.shape[1]
    info = plsc.get_sparse_core_info()
    nw = info.num_cores * info.num_subcores
    bpw = n_out // nw
    nch = bpw // GCH
    mesh = plsc.VectorSubcoreMesh(core_axis_name="c", subcore_axis_name="s")

    @functools.partial(
        pl.kernel, mesh=mesh,
        out_type=jax.ShapeDtypeStruct((n_out, dim), jnp.float32),
        scratch_types=[
            pltpu.VMEM((GCH,), jnp.int32),
            pltpu.VMEM((GCH, dim), jnp.float32),
            pltpu.SemaphoreType.DMA,
        ],
    )
    def gather_k(idx_hbm, table_hbm, out_hbm, idx_v, rows_v, sem):
        wid = lax.axis_index("s") * info.num_cores + lax.axis_index("c")
        base = wid * bpw
        for c in range(nch):
            off = base + c * GCH
            pltpu.sync_copy(idx_hbm.at[pl.ds(off, GCH)], idx_v)
            pltpu.async_copy(table_hbm.at[idx_v], rows_v, sem).wait()
            pltpu.sync_copy(rows_v, out_hbm.at[pl.ds(off, GCH)])

    return gather_k(idx, table)


def _sc_scatter_rows(idx, rows, n_out):
    n_in, dim = rows.shape
    info = plsc.get_sparse_core_info()
    nw = info.num_cores * info.num_subcores
    bpw = n_in // nw
    nch = bpw // GCH
    mesh = plsc.VectorSubcoreMesh(core_axis_name="c", subcore_axis_name="s")

    @functools.partial(
        pl.kernel, mesh=mesh,
        out_type=jax.ShapeDtypeStruct((n_out, dim), jnp.float32),
        scratch_types=[
            pltpu.VMEM((GCH,), jnp.int32),
            pltpu.VMEM((GCH, dim), jnp.float32),
            pltpu.SemaphoreType.DMA,
        ],
    )
    def scatter_k(idx_hbm, rows_hbm, out_hbm, idx_v, rows_v, sem):
        wid = lax.axis_index("s") * info.num_cores + lax.axis_index("c")
        base = wid * bpw
        for c in range(nch):
            off = base + c * GCH
            pltpu.sync_copy(idx_hbm.at[pl.ds(off, GCH)], idx_v)
            pltpu.sync_copy(rows_hbm.at[pl.ds(off, GCH)], rows_v)
            pltpu.async_copy(rows_v, out_hbm.at[idx_v], sem).wait()

    return scatter_k(idx, rows)


def _experts_body(be_ref, nu_ref, xg_ref, w1_ref, b1_ref, w2_ref, b2_ref,
                  out_ref):
    @pl.when(pl.program_id(0) * BT < nu_ref[0])
    def _work():
        x = xg_ref[...]
        h = _gelu(jax.lax.dot_general(x, w1_ref[0], (((1,), (0,)), ((), ())),
                                      preferred_element_type=jnp.float32)
                  + b1_ref[0])
        out_ref[...] = jax.lax.dot_general(h, w2_ref[0],
                                           (((1,), (0,)), ((), ())),
                                           preferred_element_type=jnp.float32)\
            + b2_ref[0]


def _run_experts(xg, block_expert, n_used, e_fc1_w, e_fc1_b, e_fc2_w,
                 e_fc2_b):
    npad, dim = xg.shape
    e, _, hid = e_fc1_w.shape
    nblk = npad // BT
    grid_spec = pltpu.PrefetchScalarGridSpec(
        num_scalar_prefetch=2,
        grid=(nblk,),
        in_specs=[
            pl.BlockSpec((BT, dim), lambda i, be, nu: (i, 0)),
            pl.BlockSpec((1, dim, hid), lambda i, be, nu: (be[i], 0, 0)),
            pl.BlockSpec((1, 1, hid), lambda i, be, nu: (be[i], 0, 0)),
            pl.BlockSpec((1, hid, dim), lambda i, be, nu: (be[i], 0, 0)),
            pl.BlockSpec((1, 1, dim), lambda i, be, nu: (be[i], 0, 0)),
        ],
        out_specs=pl.BlockSpec((BT, dim), lambda i, be, nu: (i, 0)),
    )
    return pl.pallas_call(
        _experts_body,
        grid_spec=grid_spec,
        out_shape=jax.ShapeDtypeStruct((npad, dim), jnp.float32),
    )(block_expert, n_used, xg, e_fc1_w, e_fc1_b.reshape(e, 1, hid), e_fc2_w,
      e_fc2_b.reshape(e, 1, dim))


def _combine_body(sha_ref, shb_ref, yg_ref, g_ref, out_ref):
    i = pl.program_id(0)
    nh = pl.num_programs(0) // 2
    sh = jnp.where(i < nh, sha_ref[...], shb_ref[...])
    out_ref[...] = sh + yg_ref[...] * g_ref[...]


def _run_combine(shared_a, shared_b, yg, top_gate2):
    t, dim = yg.shape
    nb = t // RB
    nh = nb // 2
    return pl.pallas_call(
        _combine_body,
        grid=(nb,),
        in_specs=[
            pl.BlockSpec((RB, dim), lambda i: (jnp.minimum(i, nh - 1), 0)),
            pl.BlockSpec((RB, dim),
                         lambda i: (jnp.clip(i - nh, 0, nh - 1), 0)),
            pl.BlockSpec((RB, dim), lambda i: (i, 0)),
            pl.BlockSpec((RB, 1), lambda i: (i, 0)),
        ],
        out_specs=pl.BlockSpec((RB, dim), lambda i: (i, 0)),
        out_shape=jax.ShapeDtypeStruct((t, dim), jnp.float32),
    )(shared_a, shared_b, yg, top_gate2)


def _shared_body(x_ref, w1_ref, b1_ref, w2_ref, b2_ref, out_ref):
    x = x_ref[...]
    acc = jnp.zeros_like(x)
    ns = w1_ref.shape[0]
    for i in range(ns):
        h = _gelu(jax.lax.dot_general(x, w1_ref[i], (((1,), (0,)), ((), ())),
                                      preferred_element_type=jnp.float32)
                  + b1_ref[i])
        acc = acc + jax.lax.dot_general(h, w2_ref[i], (((1,), (0,)), ((), ())),
                                        preferred_element_type=jnp.float32) \
            + b2_ref[i]
    out_ref[...] = acc


def _run_shared(x_flat, s_fc1_w, s_fc1_b, s_fc2_w, s_fc2_b, blk_off, nb):
    t, dim = x_flat.shape
    ns, _, hid = s_fc1_w.shape
    return pl.pallas_call(
        _shared_body,
        grid=(nb,),
        in_specs=[
            pl.BlockSpec((RB, dim), lambda i: (i + blk_off, 0)),
            pl.BlockSpec((ns, dim, hid), lambda i: (0, 0, 0)),
            pl.BlockSpec((ns, 1, hid), lambda i: (0, 0, 0)),
            pl.BlockSpec((ns, hid, dim), lambda i: (0, 0, 0)),
            pl.BlockSpec((ns, 1, dim), lambda i: (0, 0, 0)),
        ],
        out_specs=pl.BlockSpec((RB, dim), lambda i: (i, 0)),
        out_shape=jax.ShapeDtypeStruct((nb * RB, dim), jnp.float32),
    )(x_flat, s_fc1_w, s_fc1_b.reshape(ns, 1, hid), s_fc2_w,
      s_fc2_b.reshape(ns, 1, dim))


def kernel(x, s_fc1_w, s_fc1_b, s_fc2_w, s_fc2_b, e_fc1_w, e_fc1_b, e_fc2_w,
           e_fc2_b, r_w, r_b):
    b, hgt, wid_, c = x.shape
    t = b * hgt * wid_
    e = r_w.shape[-1]
    npad = t + e * BT
    x_flat = x.reshape(t, c)

    top_idx2, top_gate2, rank2, stats = _run_router(x_flat, r_w, r_b)

    dest, block_expert, n_used, balance = _run_prep(top_idx2, rank2, stats,
                                                    npad // BT)

    nb = t // RB
    xg = _sc_scatter_rows(dest, x_flat, npad)
    shared_a = _run_shared(x_flat, s_fc1_w, s_fc1_b, s_fc2_w, s_fc2_b,
                           0, nb // 2)
    n_used, shared_a = jax.lax.optimization_barrier((n_used, shared_a))
    y_pad = _run_experts(xg, block_expert, n_used, e_fc1_w, e_fc1_b, e_fc2_w,
                         e_fc2_b)
    shared_b = _run_shared(x_flat, s_fc1_w, s_fc1_b, s_fc2_w, s_fc2_b,
                           nb // 2, nb - nb // 2)
    yg = _sc_gather_rows(dest, y_pad, t)

    out = _run_combine(shared_a, shared_b, yg, top_gate2)
    return out.reshape(b, hgt, wid_, c), balance
